# baseline (speedup 1.0000x reference)
import functools

import jax
import jax.numpy as jnp
from jax import lax
from jax.experimental import pallas as pl
from jax.experimental.pallas import tpu as pltpu
from jax.experimental.pallas import tpu_sc as plsc

D = 16
SCALE = 4.0
CHUNK = 512
NBUF = 5
JT, IT = 8, 128
RUN = JT * IT

_params = pltpu.CompilerParams(
    use_tc_tiling_on_sc=False, needs_layout_passes=False)


def _scale_table(tbl_tiled, vp, nc, ns):
    nw = nc * ns
    vt = vp // IT
    band = vt * RUN
    nch = vt // 2
    cw = 2 * RUN
    trips = -(-nch // nw)
    trips += trips % 2
    mesh = plsc.VectorSubcoreMesh(core_axis_name="c", subcore_axis_name="s")

    @functools.partial(
        pl.kernel,
        mesh=mesh,
        out_type=jax.ShapeDtypeStruct((vp, D), jnp.float32),
        scratch_types=(
            [pltpu.VMEM((2 * cw,), jnp.float32) for _ in range(2)]
            + [pltpu.VMEM((2 * IT, D), jnp.float32) for _ in range(2)]
            + [pltpu.SemaphoreType.DMA for _ in range(4)]
        ),
        compiler_params=_params,
    )
    def kern(t_hbm, out_hbm, pb0, pb1, sb0, sb1, ps0, ps1, os0, os1):
        pbuf, sbuf = (pb0, pb1), (sb0, sb1)
        psem, osem = (ps0, ps1), (os0, os1)
        wid = lax.axis_index("s") * nc + lax.axis_index("c")
        lane = lax.iota(jnp.int32, D)

        def tile_copies(n, b):
            off = pl.multiple_of(n * cw, cw)
            off1 = pl.multiple_of(band + n * cw, cw)
            return [
                pltpu.make_async_copy(
                    t_hbm.at[pl.ds(off, cw)],
                    pbuf[b].at[pl.ds(0, cw)], psem[b]),
                pltpu.make_async_copy(
                    t_hbm.at[pl.ds(off1, cw)],
                    pbuf[b].at[pl.ds(cw, cw)], psem[b]),
            ]

        def store_copy(n, b):
            return pltpu.make_async_copy(
                sbuf[b], out_hbm.at[pl.ds(n * 2 * IT, 2 * IT), :], osem[b])

        for c in tile_copies(wid, 0):
            c.start()

        def outer(k2, carry):
            for b in range(2):
                k = k2 * 2 + b
                n = k * nw + wid
                np1 = (k + 1) * nw + wid

                @pl.when(np1 < nch)
                def _():
                    for c in tile_copies(np1, 1 - b):
                        c.start()

                @pl.when(jnp.logical_and(k >= 2, (k - 2) * nw + wid < nch))
                def _():
                    store_copy((k - 2) * nw + wid, b).wait()

                @pl.when(n < nch)
                def _():
                    for c in tile_copies(n, b):
                        c.wait()

                    pb = pbuf[b]
                    sb = sbuf[b]

                    @plsc.parallel_loop(0, 2 * IT, 1, unroll=8)
                    def _(q):
                        dq = (q >> 3) & (JT - 1)
                        d = ((q >> 7) << 3) + dq
                        off = (((q >> 6) & 3) << 10) + (dq << 7) + ((q & 7) << 4)
                        v = pb[pl.ds(pl.multiple_of(off, D), D)]
                        vv = (((q >> 6) & 1) << 7) + ((q & 7) << 4) + lane
                        vd = (d + vv) & (D - 1)
                        plsc.store_scatter(sb, [vv, vd], v)

                    @plsc.parallel_loop(0, 2 * IT, 1, unroll=8)
                    def _(c):
                        perm = (lane + c) & (D - 1)
                        x = sb[c].at[perm].get(mode="promise_in_bounds")
                        sb[c] = x * SCALE

                    store_copy(n, b).start()
            return carry

        lax.fori_loop(0, trips // 2, outer, 0)

        for b in range(2):
            n_last = (trips - 2 + b) * nw + wid

            @pl.when(n_last < nch)
            def _():
                store_copy(n_last, b).wait()

    return kern(tbl_tiled)


def _gather(idx_lin, table_rm, n_i, n_j, nc, ns):
    B = idx_lin.shape[0]
    ib = n_i // IT
    nw = nc * ns
    per_w = B // (nw * CHUNK)
    assert per_w * nw * CHUNK == B and per_w % NBUF == 0
    JH = CHUNK // IT
    NRUN = JH * D // JT
    mesh = plsc.VectorSubcoreMesh(core_axis_name="c", subcore_axis_name="s")

    @functools.partial(
        pl.kernel,
        mesh=mesh,
        out_type=jax.ShapeDtypeStruct((B * D,), jnp.float32),
        scratch_types=(
            [pltpu.VMEM((CHUNK,), jnp.int32) for _ in range(NBUF)]
            + [pltpu.VMEM((CHUNK, D), jnp.float32) for _ in range(NBUF)]
            + [pltpu.VMEM((CHUNK * D,), jnp.float32) for _ in range(NBUF)]
            + [pltpu.SemaphoreType.DMA for _ in range(3 * NBUF)]
        ),
        compiler_params=_params,
    )
    def kern(idx_hbm, table_hbm, out_hbm, *refs):
        idxb = refs[0:NBUF]
        rowsb = refs[NBUF:2 * NBUF]
        obufb = refs[2 * NBUF:3 * NBUF]
        isem = refs[3 * NBUF:4 * NBUF]
        gsem = refs[4 * NBUF:5 * NBUF]
        osem = refs[5 * NBUF:6 * NBUF]

        wid = lax.axis_index("s") * nc + lax.axis_index("c")
        g0 = wid * per_w
        lane = lax.iota(jnp.int32, D)

        def idx_copy(g, b):
            return pltpu.make_async_copy(
                idx_hbm.at[pl.ds((g0 + g) * CHUNK, CHUNK)], idxb[b], isem[b])

        def gather_copy(b):
            return pltpu.make_async_copy(
                table_hbm.at[idxb[b]], rowsb[b], gsem[b])

        def store_copies(g, b):
            G = g0 + g
            j0 = ((G >> 8) << 3) + ((G & 1) << 2)
            ihi = (G >> 1) & (ib - 1)
            out = []
            for r in range(NRUN):
                jl, dh = r >> 1, r & 1
                base = (((((j0 + jl) << 1) + dh) * ib + ihi) << 10)
                base = pl.multiple_of(base, RUN)
                out.append(pltpu.make_async_copy(
                    obufb[b].at[pl.ds(r * RUN, RUN)],
                    out_hbm.at[pl.ds(base, RUN)], osem[b]))
            return out

        for c in range(NBUF - 1):
            idx_copy(c, c).start()
        for c in range(NBUF - 2):
            idx_copy(c, c).wait()
            gather_copy(c).start()

        def outer(ko, carry):
            for b in range(NBUF):
                g = ko * NBUF + b

                h1 = g + (NBUF - 1)
                b1 = (b + NBUF - 1) % NBUF

                @pl.when(h1 < per_w)
                def _():
                    idx_copy(h1, b1).start()

                h2 = g + (NBUF - 2)
                b2 = (b + NBUF - 2) % NBUF

                @pl.when(jnp.logical_and(h2 < per_w, h2 >= NBUF))
                def _():
                    for c in store_copies(h2 - NBUF, b2):
                        c.wait()

                @pl.when(h2 < per_w)
                def _():
                    idx_copy(h2, b2).wait()
                    gather_copy(b2).start()

                gather_copy(b).wait()
                rb = rowsb[b]
                ob = obufb[b]

                @plsc.parallel_loop(0, CHUNK, 1, unroll=8)
                def _(p):
                    perm = (lane - p) & (D - 1)
                    x = rb[p].at[perm].get(mode="promise_in_bounds")
                    rb[p] = x

                @plsc.parallel_loop(0, CHUNK, 1, unroll=16)
                def _(q):
                    jlo = q >> 7
                    ilo0 = ((q >> 4) & 7) << 4
                    d = q & (D - 1)
                    p0 = (jlo << 7) + ilo0
                    vp = p0 + lane
                    vd = (d + vp) & (D - 1)
                    v = plsc.load_gather(rb, [vp, vd])
                    off = (jlo << 11) + ((d >> 3) << 10) + ((d & 7) << 7) + ilo0
                    ob[pl.ds(pl.multiple_of(off, D), D)] = v

                for c in store_copies(g, b):
                    c.start()
            return carry

        lax.fori_loop(0, per_w // NBUF, outer, 0)

        for b in range(NBUF):
            for c in store_copies(per_w - NBUF + b, b):
                c.wait()

    return kern(idx_lin, table_rm)


def kernel(inp, emb_table):
    n_i, n_j = inp.shape
    B = n_i * n_j
    jb, ib = n_j // JT, n_i // IT
    assert jb * JT == n_j and ib * IT == n_i

    idx_lin = (
        jnp.transpose(inp)
        .reshape(jb, JT, ib, IT)
        .transpose(0, 2, 1, 3)
        .reshape(B)
        .astype(jnp.int32)
    )

    info = plsc.get_sparse_core_info()
    nc, ns = info.num_cores, info.num_subcores

    V = emb_table.shape[0]
    vp = -(-V // (2 * IT)) * (2 * IT)
    padded = jnp.pad(emb_table, ((0, vp - V), (0, 0)))
    tbl_tiled = (
        jnp.transpose(padded)
        .reshape(D // JT, JT, vp // IT, IT)
        .transpose(0, 2, 1, 3)
        .reshape(D * vp)
    )

    table_rm = _scale_table(tbl_tiled, vp, nc, ns)
    out_flat = _gather(idx_lin, table_rm, n_i, n_j, nc, ns)

    return (out_flat.reshape(n_j, D // JT, ib, JT, IT)
            .transpose(2, 4, 0, 1, 3).reshape(n_i, n_j, D))


# speedup vs baseline: 1.0116x; 1.0116x over previous
import functools

import jax
import jax.numpy as jnp
from jax import lax
from jax.experimental import pallas as pl
from jax.experimental.pallas import tpu as pltpu
from jax.experimental.pallas import tpu_sc as plsc

D = 16
SCALE = 4.0
CHUNK = 512
NBUF = 5
JT, IT = 8, 128
RUN = JT * IT

_params = pltpu.CompilerParams(
    use_tc_tiling_on_sc=False, needs_layout_passes=False)


def _scale_table(tbl_tiled, vp, nc, ns):
    nw = nc * ns
    vt = vp // IT
    band = vt * RUN
    nch = vt // 2
    cw = 2 * RUN
    trips = -(-nch // nw)
    trips += trips % 2
    mesh = plsc.VectorSubcoreMesh(core_axis_name="c", subcore_axis_name="s")

    @functools.partial(
        pl.kernel,
        mesh=mesh,
        out_type=jax.ShapeDtypeStruct((vp, D), jnp.float32),
        scratch_types=(
            [pltpu.VMEM((2 * cw,), jnp.float32) for _ in range(2)]
            + [pltpu.VMEM((2 * IT, D), jnp.float32) for _ in range(2)]
            + [pltpu.SemaphoreType.DMA for _ in range(4)]
        ),
        compiler_params=_params,
    )
    def kern(t_hbm, out_hbm, pb0, pb1, sb0, sb1, ps0, ps1, os0, os1):
        pbuf, sbuf = (pb0, pb1), (sb0, sb1)
        psem, osem = (ps0, ps1), (os0, os1)
        wid = lax.axis_index("s") * nc + lax.axis_index("c")
        lane = lax.iota(jnp.int32, D)

        def tile_copies(n, b):
            off = pl.multiple_of(n * cw, cw)
            off1 = pl.multiple_of(band + n * cw, cw)
            return [
                pltpu.make_async_copy(
                    t_hbm.at[pl.ds(off, cw)],
                    pbuf[b].at[pl.ds(0, cw)], psem[b]),
                pltpu.make_async_copy(
                    t_hbm.at[pl.ds(off1, cw)],
                    pbuf[b].at[pl.ds(cw, cw)], psem[b]),
            ]

        def store_copy(n, b):
            return pltpu.make_async_copy(
                sbuf[b], out_hbm.at[pl.ds(n * 2 * IT, 2 * IT), :], osem[b])

        for c in tile_copies(wid, 0):
            c.start()

        def outer(k2, carry):
            for b in range(2):
                k = k2 * 2 + b
                n = k * nw + wid
                np1 = (k + 1) * nw + wid

                @pl.when(np1 < nch)
                def _():
                    for c in tile_copies(np1, 1 - b):
                        c.start()

                @pl.when(jnp.logical_and(k >= 2, (k - 2) * nw + wid < nch))
                def _():
                    store_copy((k - 2) * nw + wid, b).wait()

                @pl.when(n < nch)
                def _():
                    for c in tile_copies(n, b):
                        c.wait()

                    pb = pbuf[b]
                    sb = sbuf[b]

                    @plsc.parallel_loop(0, 2 * IT, 1, unroll=16)
                    def _(q):
                        dq = (q >> 3) & (JT - 1)
                        d = ((q >> 7) << 3) + dq
                        off = (((q >> 6) & 3) << 10) + (dq << 7) + ((q & 7) << 4)
                        v = pb[pl.ds(pl.multiple_of(off, D), D)]
                        vv = (((q >> 6) & 1) << 7) + ((q & 7) << 4) + lane
                        vd = (d + vv) & (D - 1)
                        plsc.store_scatter(sb, [vv, vd], v)

                    @plsc.parallel_loop(0, 2 * IT, 1, unroll=16)
                    def _(c):
                        perm = (lane + c) & (D - 1)
                        x = sb[c].at[perm].get(mode="promise_in_bounds")
                        sb[c] = x * SCALE

                    store_copy(n, b).start()
            return carry

        lax.fori_loop(0, trips // 2, outer, 0)

        for b in range(2):
            n_last = (trips - 2 + b) * nw + wid

            @pl.when(n_last < nch)
            def _():
                store_copy(n_last, b).wait()

    return kern(tbl_tiled)


def _gather(idx_lin, table_rm, n_i, n_j, nc, ns):
    B = idx_lin.shape[0]
    ib = n_i // IT
    nw = nc * ns
    per_w = B // (nw * CHUNK)
    assert per_w * nw * CHUNK == B and per_w % NBUF == 0
    JH = CHUNK // IT
    NRUN = JH * D // JT
    mesh = plsc.VectorSubcoreMesh(core_axis_name="c", subcore_axis_name="s")

    @functools.partial(
        pl.kernel,
        mesh=mesh,
        out_type=jax.ShapeDtypeStruct((B * D,), jnp.float32),
        scratch_types=(
            [pltpu.VMEM((CHUNK,), jnp.int32) for _ in range(NBUF)]
            + [pltpu.VMEM((CHUNK, D), jnp.float32) for _ in range(NBUF)]
            + [pltpu.VMEM((CHUNK * D,), jnp.float32) for _ in range(NBUF)]
            + [pltpu.SemaphoreType.DMA for _ in range(3 * NBUF)]
        ),
        compiler_params=_params,
    )
    def kern(idx_hbm, table_hbm, out_hbm, *refs):
        idxb = refs[0:NBUF]
        rowsb = refs[NBUF:2 * NBUF]
        obufb = refs[2 * NBUF:3 * NBUF]
        isem = refs[3 * NBUF:4 * NBUF]
        gsem = refs[4 * NBUF:5 * NBUF]
        osem = refs[5 * NBUF:6 * NBUF]

        wid = lax.axis_index("s") * nc + lax.axis_index("c")
        g0 = wid * per_w
        lane = lax.iota(jnp.int32, D)

        def idx_copy(g, b):
            return pltpu.make_async_copy(
                idx_hbm.at[pl.ds((g0 + g) * CHUNK, CHUNK)], idxb[b], isem[b])

        def gather_copy(b):
            return pltpu.make_async_copy(
                table_hbm.at[idxb[b]], rowsb[b], gsem[b])

        def store_copies(g, b):
            G = g0 + g
            j0 = ((G >> 8) << 3) + ((G & 1) << 2)
            ihi = (G >> 1) & (ib - 1)
            out = []
            for r in range(NRUN):
                jl, dh = r >> 1, r & 1
                base = (((((j0 + jl) << 1) + dh) * ib + ihi) << 10)
                base = pl.multiple_of(base, RUN)
                out.append(pltpu.make_async_copy(
                    obufb[b].at[pl.ds(r * RUN, RUN)],
                    out_hbm.at[pl.ds(base, RUN)], osem[b]))
            return out

        for c in range(NBUF - 1):
            idx_copy(c, c).start()
        for c in range(NBUF - 2):
            idx_copy(c, c).wait()
            gather_copy(c).start()

        def outer(ko, carry):
            for b in range(NBUF):
                g = ko * NBUF + b

                h1 = g + (NBUF - 1)
                b1 = (b + NBUF - 1) % NBUF

                @pl.when(h1 < per_w)
                def _():
                    idx_copy(h1, b1).start()

                h2 = g + (NBUF - 2)
                b2 = (b + NBUF - 2) % NBUF

                @pl.when(jnp.logical_and(h2 < per_w, h2 >= NBUF))
                def _():
                    for c in store_copies(h2 - NBUF, b2):
                        c.wait()

                @pl.when(h2 < per_w)
                def _():
                    idx_copy(h2, b2).wait()
                    gather_copy(b2).start()

                gather_copy(b).wait()
                rb = rowsb[b]
                ob = obufb[b]

                @plsc.parallel_loop(0, CHUNK, 1, unroll=16)
                def _(p):
                    perm = (lane - p) & (D - 1)
                    x = rb[p].at[perm].get(mode="promise_in_bounds")
                    rb[p] = x

                @plsc.parallel_loop(0, CHUNK, 1, unroll=16)
                def _(q):
                    jlo = q >> 7
                    ilo0 = ((q >> 4) & 7) << 4
                    d = q & (D - 1)
                    p0 = (jlo << 7) + ilo0
                    vp = p0 + lane
                    vd = (d + vp) & (D - 1)
                    v = plsc.load_gather(rb, [vp, vd])
                    off = (jlo << 11) + ((d >> 3) << 10) + ((d & 7) << 7) + ilo0
                    ob[pl.ds(pl.multiple_of(off, D), D)] = v

                for c in store_copies(g, b):
                    c.start()
            return carry

        lax.fori_loop(0, per_w // NBUF, outer, 0)

        for b in range(NBUF):
            for c in store_copies(per_w - NBUF + b, b):
                c.wait()

    return kern(idx_lin, table_rm)


def kernel(inp, emb_table):
    n_i, n_j = inp.shape
    B = n_i * n_j
    jb, ib = n_j // JT, n_i // IT
    assert jb * JT == n_j and ib * IT == n_i

    idx_lin = (
        jnp.transpose(inp)
        .reshape(jb, JT, ib, IT)
        .transpose(0, 2, 1, 3)
        .reshape(B)
        .astype(jnp.int32)
    )

    info = plsc.get_sparse_core_info()
    nc, ns = info.num_cores, info.num_subcores

    V = emb_table.shape[0]
    vp = -(-V // (2 * IT)) * (2 * IT)
    padded = jnp.pad(emb_table, ((0, vp - V), (0, 0)))
    tbl_tiled = (
        jnp.transpose(padded)
        .reshape(D // JT, JT, vp // IT, IT)
        .transpose(0, 2, 1, 3)
        .reshape(D * vp)
    )

    table_rm = _scale_table(tbl_tiled, vp, nc, ns)
    out_flat = _gather(idx_lin, table_rm, n_i, n_j, nc, ns)

    return (out_flat.reshape(n_j, D // JT, ib, JT, IT)
            .transpose(2, 4, 0, 1, 3).reshape(n_i, n_j, D))
